# speedup vs baseline: 2.9682x; 2.9682x over previous
import functools

import jax
import jax.numpy as jnp
from jax import lax
from jax.experimental import pallas as pl
from jax.experimental.pallas import tpu as pltpu
from jax.experimental.pallas import tpu_sc as plsc

_NUM_EMBEDDINGS = 100000
_DIM = 128
_BATCH = 4096 * 50
_NUM_WORKERS = 32
_ROWS_PER_WORKER = _BATCH // _NUM_WORKERS
_CHUNK = 128
_NUM_CHUNKS = _ROWS_PER_WORKER // _CHUNK


_mesh = plsc.VectorSubcoreMesh(core_axis_name="c", subcore_axis_name="s")


@functools.partial(
    pl.kernel,
    mesh=_mesh,
    out_type=jax.ShapeDtypeStruct((_BATCH, _DIM), jnp.float32),
    scratch_types=[
        pltpu.VMEM((_NUM_CHUNKS, _CHUNK), jnp.int32),
        pltpu.VMEM((_CHUNK, _DIM), jnp.float32),
        pltpu.SemaphoreType.DMA,
    ],
)
def _gather_kernel(table_hbm, idx_hbm, out_hbm, idx_v, rows_v, gsem):
    wid = lax.axis_index("s") * 2 + lax.axis_index("c")
    base = wid * _ROWS_PER_WORKER
    pltpu.sync_copy(idx_hbm.at[wid], idx_v)

    def body(c, _):
        pltpu.async_copy(table_hbm.at[idx_v.at[c]], rows_v, gsem).wait()
        pltpu.sync_copy(rows_v, out_hbm.at[pl.ds(base + c * _CHUNK, _CHUNK)])
        return ()

    lax.fori_loop(0, _NUM_CHUNKS, body, ())


def kernel(x, weight):
    idx = x.reshape(_NUM_WORKERS, _NUM_CHUNKS, _CHUNK).astype(jnp.int32)
    out = _gather_kernel(weight, idx)
    return out.reshape(x.shape[0], x.shape[1], _DIM)


# speedup vs baseline: 3.1251x; 1.0529x over previous
import functools

import jax
import jax.numpy as jnp
from jax import lax
from jax.experimental import pallas as pl
from jax.experimental.pallas import tpu as pltpu
from jax.experimental.pallas import tpu_sc as plsc

_NUM_EMBEDDINGS = 100000
_DIM = 128
_BATCH = 4096 * 50
_NUM_WORKERS = 32
_ROWS_PER_WORKER = _BATCH // _NUM_WORKERS
_CHUNK = 128
_NUM_CHUNKS = _ROWS_PER_WORKER // _CHUNK


_mesh = plsc.VectorSubcoreMesh(core_axis_name="c", subcore_axis_name="s")


@functools.partial(
    pl.kernel,
    mesh=_mesh,
    out_type=jax.ShapeDtypeStruct((_BATCH, _DIM), jnp.float32),
    scratch_types=[
        pltpu.VMEM((_NUM_CHUNKS, _CHUNK), jnp.int32),
        pltpu.VMEM((_CHUNK, _DIM), jnp.float32),
        pltpu.VMEM((_CHUNK, _DIM), jnp.float32),
        pltpu.SemaphoreType.DMA,
        pltpu.SemaphoreType.DMA,
        pltpu.SemaphoreType.DMA,
        pltpu.SemaphoreType.DMA,
    ],
)
def _gather_kernel(table_hbm, idx_hbm, out_hbm, idx_v, buf0, buf1,
                   gsem0, gsem1, osem0, osem1):
    wid = lax.axis_index("s") * 2 + lax.axis_index("c")
    base = wid * _ROWS_PER_WORKER
    bufs = (buf0, buf1)
    gsems = (gsem0, gsem1)
    osems = (osem0, osem1)

    pltpu.sync_copy(idx_hbm.at[wid], idx_v)

    def gather(c, b):
        return pltpu.make_async_copy(
            table_hbm.at[idx_v.at[c]], bufs[b], gsems[b])

    def out_copy(c, b):
        return pltpu.make_async_copy(
            bufs[b], out_hbm.at[pl.ds(base + c * _CHUNK, _CHUNK)], osems[b])

    gather(0, 0).start()
    gather(0, 0).wait()
    gather(1, 1).start()
    out_copy(0, 0).start()

    def body(g, _):
        c = 1 + 2 * g
        gather(c, 1).wait()
        out_copy(c - 1, 0).wait()
        gather(c + 1, 0).start()
        out_copy(c, 1).start()
        gather(c + 1, 0).wait()
        out_copy(c, 1).wait()
        gather(c + 2, 1).start()
        out_copy(c + 1, 0).start()
        return ()

    lax.fori_loop(0, (_NUM_CHUNKS - 2) // 2, body, ())

    last = _NUM_CHUNKS - 1
    gather(last, 1).wait()
    out_copy(last - 1, 0).wait()
    out_copy(last, 1).start()
    out_copy(last, 1).wait()


def kernel(x, weight):
    idx = x.reshape(_NUM_WORKERS, _NUM_CHUNKS, _CHUNK).astype(jnp.int32)
    out = _gather_kernel(weight, idx)
    return out.reshape(x.shape[0], x.shape[1], _DIM)


# speedup vs baseline: 5.8747x; 1.8798x over previous
import functools

import jax
import jax.numpy as jnp
from jax import lax
from jax.experimental import pallas as pl
from jax.experimental.pallas import tpu as pltpu
from jax.experimental.pallas import tpu_sc as plsc

_NUM_EMBEDDINGS = 100000
_DIM = 128
_B = 4096
_S = 50
_NUM_WORKERS = 32
_B_PER_W = _B // _NUM_WORKERS
_NBUF = 4


_mesh = plsc.VectorSubcoreMesh(core_axis_name="c", subcore_axis_name="s")


@functools.partial(
    pl.kernel,
    mesh=_mesh,
    out_type=jax.ShapeDtypeStruct((_B, _S, _DIM), jnp.float32),
    scratch_types=[
        pltpu.VMEM((_B_PER_W, _S), jnp.int32),
        [pltpu.VMEM((_S, _DIM), jnp.float32) for _ in range(_NBUF)],
        [pltpu.SemaphoreType.DMA for _ in range(_NBUF)],
        [pltpu.SemaphoreType.DMA for _ in range(_NBUF)],
    ],
    compiler_params=pltpu.CompilerParams(use_tc_tiling_on_sc=True),
)
def _gather_kernel(table_hbm, idx_hbm, out_hbm, idx_v, bufs, gsems, osems):
    wid = lax.axis_index("s") * 2 + lax.axis_index("c")
    base = wid * _B_PER_W

    pltpu.sync_copy(idx_hbm.at[pl.ds(base, _B_PER_W)], idx_v)

    def gather(c, b):
        return pltpu.make_async_copy(
            table_hbm.at[idx_v.at[c]], bufs[b], gsems[b])

    def out_copy(c, b):
        return pltpu.make_async_copy(bufs[b], out_hbm.at[base + c], osems[b])

    for c in range(_NBUF):
        gather(c, c).start()
    gather(0, 0).wait()
    out_copy(0, 0).start()

    def step(c, jb):
        gather(c, jb).wait()
        out_copy(c - 1, (jb - 1) % _NBUF).wait()
        gather(c + (_NBUF - 1), (jb - 1) % _NBUF).start()
        out_copy(c, jb).start()

    def body(g, _):
        c0 = 1 + _NBUF * g
        for j in range(_NBUF):
            step(c0 + j, (1 + j) % _NBUF)
        return ()

    n_main = _B_PER_W - _NBUF
    lax.fori_loop(0, n_main // _NBUF, body, ())

    for c in range(_B_PER_W - (_NBUF - 1), _B_PER_W):
        gather(c, c % _NBUF).wait()
        out_copy(c - 1, (c - 1) % _NBUF).wait()
        out_copy(c, c % _NBUF).start()
    out_copy(_B_PER_W - 1, (_B_PER_W - 1) % _NBUF).wait()


def kernel(x, weight):
    return _gather_kernel(weight, x.astype(jnp.int32))


# speedup vs baseline: 5.9638x; 1.0152x over previous
import functools

import jax
import jax.numpy as jnp
from jax import lax
from jax.experimental import pallas as pl
from jax.experimental.pallas import tpu as pltpu
from jax.experimental.pallas import tpu_sc as plsc

_NUM_EMBEDDINGS = 100000
_DIM = 128
_B = 4096
_S = 50
_NUM_WORKERS = 32
_B_PER_W = _B // _NUM_WORKERS
_NBUF = 8


_mesh = plsc.VectorSubcoreMesh(core_axis_name="c", subcore_axis_name="s")


@functools.partial(
    pl.kernel,
    mesh=_mesh,
    out_type=jax.ShapeDtypeStruct((_B, _S, _DIM), jnp.float32),
    scratch_types=[
        pltpu.VMEM((_B_PER_W, _S), jnp.int32),
        [pltpu.VMEM((_S, _DIM), jnp.float32) for _ in range(_NBUF)],
        [pltpu.SemaphoreType.DMA for _ in range(_NBUF)],
        [pltpu.SemaphoreType.DMA for _ in range(_NBUF)],
    ],
    compiler_params=pltpu.CompilerParams(use_tc_tiling_on_sc=True),
)
def _gather_kernel(table_hbm, idx_hbm, out_hbm, idx_v, bufs, gsems, osems):
    wid = lax.axis_index("s") * 2 + lax.axis_index("c")
    base = wid * _B_PER_W

    pltpu.sync_copy(idx_hbm.at[pl.ds(base, _B_PER_W)], idx_v)

    def gather(c, b):
        return pltpu.make_async_copy(
            table_hbm.at[idx_v.at[c]], bufs[b], gsems[b])

    def out_copy(c, b):
        return pltpu.make_async_copy(bufs[b], out_hbm.at[base + c], osems[b])

    for c in range(_NBUF):
        gather(c, c).start()
    gather(0, 0).wait()
    out_copy(0, 0).start()

    def step(c, jb):
        gather(c, jb).wait()
        out_copy(c - 1, (jb - 1) % _NBUF).wait()
        gather(c + (_NBUF - 1), (jb - 1) % _NBUF).start()
        out_copy(c, jb).start()

    def body(g, _):
        c0 = 1 + _NBUF * g
        for j in range(_NBUF):
            step(c0 + j, (1 + j) % _NBUF)
        return ()

    n_main = _B_PER_W - _NBUF
    lax.fori_loop(0, n_main // _NBUF, body, ())

    for c in range(_B_PER_W - (_NBUF - 1), _B_PER_W):
        gather(c, c % _NBUF).wait()
        out_copy(c - 1, (c - 1) % _NBUF).wait()
        out_copy(c, c % _NBUF).start()
    out_copy(_B_PER_W - 1, (_B_PER_W - 1) % _NBUF).wait()


def kernel(x, weight):
    return _gather_kernel(weight, x.astype(jnp.int32))


# speedup vs baseline: 10.6532x; 1.7863x over previous
import functools

import jax
import jax.numpy as jnp
from jax import lax
from jax.experimental import pallas as pl
from jax.experimental.pallas import tpu as pltpu
from jax.experimental.pallas import tpu_sc as plsc

_NUM_EMBEDDINGS = 100000
_DIM = 128
_B = 4096
_S = 50
_NUM_WORKERS = 32
_B_PER_W = _B // _NUM_WORKERS
_NBUF = 4


_mesh = plsc.VectorSubcoreMesh(core_axis_name="c", subcore_axis_name="s")


@functools.partial(
    pl.kernel,
    mesh=_mesh,
    out_type=jax.ShapeDtypeStruct((_S, _B, _DIM), jnp.float32),
    scratch_types=[
        pltpu.VMEM((_S, _B_PER_W), jnp.int32),
        [pltpu.VMEM((_B_PER_W, _DIM), jnp.float32) for _ in range(_NBUF)],
        [pltpu.SemaphoreType.DMA for _ in range(_NBUF)],
        [pltpu.SemaphoreType.DMA for _ in range(_NBUF)],
    ],
    compiler_params=pltpu.CompilerParams(use_tc_tiling_on_sc=True),
)
def _gather_kernel(table_hbm, idx_hbm, out_hbm, idx_v, bufs, gsems, osems):
    wid = lax.axis_index("s") * 2 + lax.axis_index("c")
    base = wid * _B_PER_W

    pltpu.sync_copy(idx_hbm.at[:, pl.ds(base, _B_PER_W)], idx_v)

    def gather(c, b):
        return pltpu.make_async_copy(
            table_hbm.at[idx_v.at[c]], bufs[b], gsems[b])

    def out_copy(c, b):
        return pltpu.make_async_copy(
            bufs[b], out_hbm.at[c, pl.ds(base, _B_PER_W)], osems[b])

    def step(c, jb, issue_gather=True):
        gather(c, jb).wait()
        out_copy(c - 1, (jb - 1) % _NBUF).wait()
        if issue_gather:
            gather(c + (_NBUF - 1), (jb - 1) % _NBUF).start()
        out_copy(c, jb).start()

    for c in range(_NBUF):
        gather(c, c).start()
    gather(0, 0).wait()
    out_copy(0, 0).start()

    def body(g, _):
        c0 = 1 + _NBUF * g
        for j in range(_NBUF):
            step(c0 + j, (1 + j) % _NBUF)
        return ()

    n_groups = (_S - 1 - (_NBUF - 1)) // _NBUF
    lax.fori_loop(0, n_groups, body, ())

    for c in range(1 + n_groups * _NBUF, _S):
        step(c, c % _NBUF, issue_gather=(c + _NBUF - 1 < _S))
    out_copy(_S - 1, (_S - 1) % _NBUF).wait()


def kernel(x, weight):
    out = _gather_kernel(weight, x.T.astype(jnp.int32))
    return out.transpose(1, 0, 2)


# speedup vs baseline: 10.7678x; 1.0108x over previous
import functools

import jax
import jax.numpy as jnp
from jax import lax
from jax.experimental import pallas as pl
from jax.experimental.pallas import tpu as pltpu
from jax.experimental.pallas import tpu_sc as plsc

_NUM_EMBEDDINGS = 100000
_DIM = 128
_B = 4096
_S = 50
_NUM_WORKERS = 32
_B_PER_W = _B // _NUM_WORKERS
_NBUF = 6


_mesh = plsc.VectorSubcoreMesh(core_axis_name="c", subcore_axis_name="s")


@functools.partial(
    pl.kernel,
    mesh=_mesh,
    out_type=jax.ShapeDtypeStruct((_S, _B, _DIM), jnp.float32),
    scratch_types=[
        pltpu.VMEM((_S, _B_PER_W), jnp.int32),
        [pltpu.VMEM((_B_PER_W, _DIM), jnp.float32) for _ in range(_NBUF)],
        [pltpu.SemaphoreType.DMA for _ in range(_NBUF)],
        [pltpu.SemaphoreType.DMA for _ in range(_NBUF)],
    ],
    compiler_params=pltpu.CompilerParams(use_tc_tiling_on_sc=True),
)
def _gather_kernel(table_hbm, idx_hbm, out_hbm, idx_v, bufs, gsems, osems):
    wid = lax.axis_index("s") * 2 + lax.axis_index("c")
    base = wid * _B_PER_W

    pltpu.sync_copy(idx_hbm.at[:, pl.ds(base, _B_PER_W)], idx_v)

    def gather(c, b):
        return pltpu.make_async_copy(
            table_hbm.at[idx_v.at[c]], bufs[b], gsems[b])

    def out_copy(c, b):
        return pltpu.make_async_copy(
            bufs[b], out_hbm.at[c, pl.ds(base, _B_PER_W)], osems[b])

    def step(c, jb, issue_gather=True):
        gather(c, jb).wait()
        out_copy(c - 1, (jb - 1) % _NBUF).wait()
        if issue_gather:
            gather(c + (_NBUF - 1), (jb - 1) % _NBUF).start()
        out_copy(c, jb).start()

    for c in range(_NBUF):
        gather(c, c).start()
    gather(0, 0).wait()
    out_copy(0, 0).start()

    def body(g, _):
        c0 = 1 + _NBUF * g
        for j in range(_NBUF):
            step(c0 + j, (1 + j) % _NBUF)
        return ()

    n_groups = (_S - 1 - (_NBUF - 1)) // _NBUF
    lax.fori_loop(0, n_groups, body, ())

    for c in range(1 + n_groups * _NBUF, _S):
        step(c, c % _NBUF, issue_gather=(c + _NBUF - 1 < _S))
    out_copy(_S - 1, (_S - 1) % _NBUF).wait()


def kernel(x, weight):
    out = _gather_kernel(weight, x.T.astype(jnp.int32))
    return out.transpose(1, 0, 2)
